# baseline (speedup 1.0000x reference)
import functools

import jax
import jax.numpy as jnp
from jax import lax
from jax.experimental import pallas as pl
from jax.experimental.pallas import tpu as pltpu
from jax.experimental.pallas import tpu_sc as plsc

N_ROWS = 1024
N_COLS = 100000
TOPK = int(0.7 * N_ROWS)

NW = 32
GPW = N_ROWS // NW

TC_BR = 64


def _sc_gather_body(xflat, fidx, tval_out, idx_v, val_v, gsem):
    wid = lax.axis_index("s") * 2 + lax.axis_index("c")
    base = wid * GPW
    pltpu.sync_copy(fidx.at[pl.ds(base, GPW)], idx_v)
    pltpu.async_copy(xflat.at[idx_v], val_v, gsem).wait()
    pltpu.sync_copy(val_v, tval_out.at[pl.ds(base, GPW)])


def _make_sc_gather():
    mesh = plsc.VectorSubcoreMesh(core_axis_name="c", subcore_axis_name="s")
    return functools.partial(
        pl.kernel,
        mesh=mesh,
        out_type=jax.ShapeDtypeStruct((N_ROWS,), jnp.float32),
        scratch_types=[
            pltpu.VMEM((GPW,), jnp.int32),
            pltpu.VMEM((GPW,), jnp.float32),
            pltpu.SemaphoreType.DMA,
        ],
    )(_sc_gather_body)


def _tc_dense_kernel(x_ref, out_ref):
    x = x_ref[...]
    m = jnp.max(x, axis=1, keepdims=True)
    s = jnp.sum(jnp.exp(x - m), axis=1, keepdims=True)
    out_ref[...] = m + jnp.log(s)


def _combine_kernel(a_ref, tval_ref, out_ref):
    loss = a_ref[...] - tval_ref[...]
    lo = jnp.min(loss)
    hi = jnp.max(loss)

    def body(_, carry):
        lo, hi = carry
        mid = 0.5 * (lo + hi)
        c = jnp.sum((loss > mid).astype(jnp.float32))
        take = c >= TOPK
        return jnp.where(take, mid, lo), jnp.where(take, hi, mid)

    lo, hi = lax.fori_loop(0, 32, body, (lo, hi))
    gt = loss > hi
    c_hi = jnp.sum(gt.astype(jnp.float32))
    s_hi = jnp.sum(jnp.where(gt, loss, 0.0))
    mean = (s_hi + (TOPK - c_hi) * hi) / TOPK
    out_ref[...] = jnp.broadcast_to(mean, (1, 1))


def kernel(input, target):
    xflat = input.reshape(-1)
    fidx = (jnp.arange(N_ROWS, dtype=jnp.int32) * N_COLS
            + target.astype(jnp.int32))
    tval = _make_sc_gather()(xflat, fidx)
    a = pl.pallas_call(
        _tc_dense_kernel,
        grid=(N_ROWS // TC_BR,),
        in_specs=[pl.BlockSpec((TC_BR, N_COLS), lambda i: (i, 0))],
        out_specs=pl.BlockSpec((TC_BR, 1), lambda i: (i, 0)),
        out_shape=jax.ShapeDtypeStruct((N_ROWS, 1), jnp.float32),
    )(input)
    out = pl.pallas_call(
        _combine_kernel,
        out_shape=jax.ShapeDtypeStruct((1, 1), jnp.float32),
    )(a, tval.reshape(N_ROWS, 1))
    return out[0, 0]


# speedup vs baseline: 2.1128x; 2.1128x over previous
import jax
import jax.numpy as jnp
from jax import lax
from jax.experimental import pallas as pl
from jax.experimental.pallas import tpu as pltpu

N_ROWS = 1024
N_COLS = 100000
BR = 64
R_BLOCKS = N_ROWS // BR
TOPK = int(0.7 * N_ROWS)


def _loss_kernel(x_ref, tgt_ref, out_ref):
    x = x_ref[...]
    m = jnp.max(x, axis=1, keepdims=True)
    s = jnp.sum(jnp.exp(x - m), axis=1, keepdims=True)
    cols = lax.broadcasted_iota(jnp.int32, x.shape, 1)
    tv = jnp.sum(jnp.where(cols == tgt_ref[...], x, 0.0), axis=1,
                 keepdims=True)
    out_ref[...] = m + jnp.log(s) - tv


def _topk_mean_kernel(loss_ref, out_ref):
    x = loss_ref[...]
    lo = jnp.min(x)
    hi = jnp.max(x)

    def body(_, carry):
        lo, hi = carry
        mid = 0.5 * (lo + hi)
        c = jnp.sum((x > mid).astype(jnp.float32))
        take = c >= TOPK
        return jnp.where(take, mid, lo), jnp.where(take, hi, mid)

    lo, hi = lax.fori_loop(0, 40, body, (lo, hi))
    gt = x > hi
    c_hi = jnp.sum(gt.astype(jnp.float32))
    s_hi = jnp.sum(jnp.where(gt, x, 0.0))
    mean = (s_hi + (TOPK - c_hi) * hi) / TOPK
    out_ref[...] = jnp.broadcast_to(mean, (1, 1))


def kernel(input, target):
    tgt = target.astype(jnp.int32).reshape(N_ROWS, 1)
    loss = pl.pallas_call(
        _loss_kernel,
        grid=(R_BLOCKS,),
        in_specs=[
            pl.BlockSpec((BR, N_COLS), lambda i: (i, 0)),
            pl.BlockSpec((BR, 1), lambda i: (i, 0)),
        ],
        out_specs=pl.BlockSpec((BR, 1), lambda i: (i, 0)),
        out_shape=jax.ShapeDtypeStruct((N_ROWS, 1), jnp.float32),
        compiler_params=pltpu.CompilerParams(
            dimension_semantics=("parallel",),
        ),
    )(input, tgt)
    out = pl.pallas_call(
        _topk_mean_kernel,
        out_shape=jax.ShapeDtypeStruct((1, 1), jnp.float32),
    )(loss.reshape(8, 128))
    return out[0, 0]
